# baseline (speedup 1.0000x reference)
import functools

import jax
import jax.numpy as jnp
from jax import lax
from jax.experimental import pallas as pl
from jax.experimental.pallas import tpu as pltpu
from jax.experimental.pallas import tpu_sc as plsc

_F = 32
_LANES = 16


def _make_fpmc(L):
    @functools.partial(
        pl.kernel,
        out_type=jax.ShapeDtypeStruct((_LANES,), jnp.float32),
        scratch_types=[
            pltpu.VMEM((L,), jnp.int32),
            pltpu.VMEM((8,), jnp.int32),
            pltpu.VMEM((8,), jnp.int32),
            pltpu.VMEM((_LANES,), jnp.float32),
            pltpu.VMEM((L, _F), jnp.float32),
            pltpu.VMEM((L, _F), jnp.float32),
            pltpu.VMEM((8, _F), jnp.float32),
            pltpu.VMEM((8, _F), jnp.float32),
            pltpu.VMEM((8, _F), jnp.float32),
            pltpu.VMEM((8, _F), jnp.float32),
            pltpu.VMEM((_LANES,), jnp.float32),
            pltpu.SemaphoreType.DMA,
        ],
        mesh=plsc.VectorSubcoreMesh(core_axis_name="c", subcore_axis_name="s"),
        compiler_params=pltpu.CompilerParams(
            needs_layout_passes=False, use_tc_tiling_on_sc=False),
    )
    def fpmc(idx_l_hbm, idx_i_hbm, idx_u_hbm, fac_hbm,
             v_il, v_li, v_ul, v_lu, v_ui, v_iu,
             out_hbm,
             idx_l_v, idx_i_v, idx_u_v, fac_v,
             rows_li, rows_lu, row_il, row_iu, row_ul, row_ui,
             res_v, sem):
        cid = lax.axis_index("c")
        sid = lax.axis_index("s")

        @pl.when(jnp.logical_and(cid == 0, sid == 0))
        def _():
            pltpu.sync_copy(idx_l_hbm, idx_l_v)
            pltpu.sync_copy(idx_i_hbm, idx_i_v)
            pltpu.sync_copy(idx_u_hbm, idx_u_v)
            pltpu.sync_copy(fac_hbm, fac_v)
            copies = [
                pltpu.async_copy(v_li.at[idx_l_v], rows_li, sem),
                pltpu.async_copy(v_lu.at[idx_l_v], rows_lu, sem),
                pltpu.async_copy(v_il.at[idx_i_v], row_il, sem),
                pltpu.async_copy(v_iu.at[idx_i_v], row_iu, sem),
                pltpu.async_copy(v_ul.at[idx_u_v], row_ul, sem),
                pltpu.async_copy(v_ui.at[idx_u_v], row_ui, sem),
            ]
            for c in copies:
                c.wait()

            half0 = pl.ds(0, _LANES)
            half1 = pl.ds(_LANES, _LANES)
            li_a = rows_li[0, half0]
            li_b = rows_li[0, half1]
            lu_a = rows_lu[0, half0]
            lu_b = rows_lu[0, half1]
            for l in range(1, L):
                li_a = li_a + rows_li[l, half0]
                li_b = li_b + rows_li[l, half1]
                lu_a = lu_a + rows_lu[l, half0]
                lu_b = lu_b + rows_lu[l, half1]

            fac = fac_v[...]
            r = (row_il[0, half0] * li_a + row_il[0, half1] * li_b
                 + row_ul[0, half0] * lu_a + row_ul[0, half1] * lu_b) * fac
            r = r + row_ui[0, half0] * row_iu[0, half0]
            r = r + row_ui[0, half1] * row_iu[0, half1]
            lanes = lax.iota(jnp.int32, _LANES)
            res_v[...] = r
            for sh in (8, 4, 2, 1):
                r = r + plsc.load_gather(res_v, [lanes ^ sh])
                res_v[...] = r
            pltpu.sync_copy(res_v, out_hbm)

    return fpmc


def kernel(u, i, t, last_basket, V_IL, V_LI, V_UL, V_LU, V_UI, V_IU):
    L = last_basket.shape[0]
    idx_l = (last_basket - 1).astype(jnp.int32)
    idx_i = jnp.full((8,), i - 1, jnp.int32)
    idx_u = jnp.full((8,), u - 1, jnp.int32)
    fac = jnp.full(
        (_LANES,),
        jnp.where(t > 0, jnp.float32(1.0 / L), jnp.float32(0.0)),
        jnp.float32,
    )
    out = _make_fpmc(L)(idx_l, idx_i, idx_u, fac,
                        V_IL, V_LI, V_UL, V_LU, V_UI, V_IU)
    return out[0]


# speedup vs baseline: 1.4044x; 1.4044x over previous
import functools

import jax
import jax.numpy as jnp
from jax import lax
from jax.experimental import pallas as pl
from jax.experimental.pallas import tpu as pltpu
from jax.experimental.pallas import tpu_sc as plsc

_F = 32
_LANES = 16


def _make_fpmc(L):
    P = 80

    @functools.partial(
        pl.kernel,
        out_type=jax.ShapeDtypeStruct((_LANES,), jnp.float32),
        scratch_types=[
            pltpu.VMEM((P,), jnp.int32),
            pltpu.VMEM((L, _F), jnp.float32),
            pltpu.VMEM((L, _F), jnp.float32),
            pltpu.VMEM((1, _F), jnp.float32),
            pltpu.VMEM((1, _F), jnp.float32),
            pltpu.VMEM((1, _F), jnp.float32),
            pltpu.VMEM((1, _F), jnp.float32),
            pltpu.VMEM((_LANES,), jnp.float32),
            pltpu.SemaphoreType.DMA,
        ],
        mesh=plsc.VectorSubcoreMesh(core_axis_name="c", subcore_axis_name="s"),
        compiler_params=pltpu.CompilerParams(needs_layout_passes=False),
    )
    def fpmc(packed_hbm,
             v_il, v_li, v_ul, v_lu, v_ui, v_iu,
             out_hbm,
             idx_v,
             rows_li, rows_lu, row_il, row_iu, row_ul, row_ui,
             res_v, sem):
        cid = lax.axis_index("c")
        sid = lax.axis_index("s")

        @pl.when(jnp.logical_and(cid == 0, sid == 0))
        def _():
            pltpu.sync_copy(packed_hbm, idx_v)
            vs = [idx_v[pl.ds(16 * b, 16)] for b in range(P // 16)]
            copies = []
            for l in range(L):
                idx = vs[l // 16][l % 16] - 1
                copies.append(pltpu.async_copy(
                    v_li.at[pl.ds(idx, 1)], rows_li.at[pl.ds(l, 1)], sem))
                copies.append(pltpu.async_copy(
                    v_lu.at[pl.ds(idx, 1)], rows_lu.at[pl.ds(l, 1)], sem))
            i0 = vs[3][8] - 1
            u0 = vs[4][0] - 1
            copies.append(pltpu.async_copy(v_il.at[pl.ds(i0, 1)], row_il, sem))
            copies.append(pltpu.async_copy(v_iu.at[pl.ds(i0, 1)], row_iu, sem))
            copies.append(pltpu.async_copy(v_ul.at[pl.ds(u0, 1)], row_ul, sem))
            copies.append(pltpu.async_copy(v_ui.at[pl.ds(u0, 1)], row_ui, sem))
            for c in copies:
                c.wait()

            half0 = pl.ds(0, _LANES)
            half1 = pl.ds(_LANES, _LANES)
            li_a = rows_li[0, half0]
            li_b = rows_li[0, half1]
            lu_a = rows_lu[0, half0]
            lu_b = rows_lu[0, half1]
            for l in range(1, L):
                li_a = li_a + rows_li[l, half0]
                li_b = li_b + rows_li[l, half1]
                lu_a = lu_a + rows_lu[l, half0]
                lu_b = lu_b + rows_lu[l, half1]

            fac_s = jnp.where(vs[4][8] > 0,
                              jnp.float32(1.0 / L), jnp.float32(0.0))
            fac = jnp.full((_LANES,), fac_s, jnp.float32)
            r = (row_il[0, half0] * li_a + row_il[0, half1] * li_b
                 + row_ul[0, half0] * lu_a + row_ul[0, half1] * lu_b) * fac
            r = r + row_ui[0, half0] * row_iu[0, half0]
            r = r + row_ui[0, half1] * row_iu[0, half1]
            lanes = lax.iota(jnp.int32, _LANES)
            res_v[...] = r
            for sh in (8, 4, 2, 1):
                r = r + plsc.load_gather(res_v, [lanes ^ sh])
                res_v[...] = r
            pltpu.sync_copy(res_v, out_hbm)

    return fpmc


def kernel(u, i, t, last_basket, V_IL, V_LI, V_UL, V_LU, V_UI, V_IU):
    L = last_basket.shape[0]
    lb = last_basket.astype(jnp.int32)
    packed = jnp.concatenate([
        lb,
        jnp.ones((56 - L,), jnp.int32),
        jnp.asarray(i, jnp.int32)[None],
        jnp.ones((7,), jnp.int32),
        jnp.asarray(u, jnp.int32)[None],
        jnp.ones((7,), jnp.int32),
        jnp.asarray(t, jnp.int32)[None],
        jnp.ones((7,), jnp.int32),
    ])
    out = _make_fpmc(L)(packed, V_IL, V_LI, V_UL, V_LU, V_UI, V_IU)
    return out[0]
